# speedup vs baseline: 2.0397x; 2.0397x over previous
import functools

import jax
import jax.numpy as jnp
from jax import lax
from jax.experimental import pallas as pl
from jax.experimental.pallas import tpu as pltpu
from jax.experimental.pallas import tpu_sc as plsc

MAX_LEN = 8192
D_MODEL = 768
BATCH = 4
SEQ = 8192
B_TOT = BATCH * SEQ
NW = 32
B_PER_W = B_TOT // NW
CHUNK = 64
NCHUNK = B_PER_W // CHUNK

_mesh = plsc.VectorSubcoreMesh(core_axis_name="c", subcore_axis_name="s")


@functools.partial(
    pl.kernel,
    mesh=_mesh,
    out_type=jax.ShapeDtypeStruct((B_TOT, D_MODEL), jnp.float32),
    scratch_types=[
        pltpu.VMEM((NCHUNK, CHUNK), jnp.int32),
        pltpu.VMEM((CHUNK, D_MODEL), jnp.float32),
        pltpu.VMEM((CHUNK, D_MODEL), jnp.float32),
        pltpu.SemaphoreType.DMA,
        pltpu.SemaphoreType.DMA,
    ],
)
def _gather_rows(idx_hbm, table_hbm, out_hbm, idx_v, buf0, buf1, sem0, sem1):
    wid = lax.axis_index("s") * 2 + lax.axis_index("c")
    base = wid * B_PER_W
    pltpu.sync_copy(idx_hbm.at[wid], idx_v)

    bufs = (buf0, buf1)
    sems = (sem0, sem1)
    copies = [None, None]
    copies[0] = pltpu.async_copy(table_hbm.at[idx_v.at[0]], bufs[0], sems[0])
    for c in range(NCHUNK):
        cur = c % 2
        nxt = (c + 1) % 2
        copies[cur].wait()
        if c + 1 < NCHUNK:
            copies[nxt] = pltpu.async_copy(
                table_hbm.at[idx_v.at[c + 1]], bufs[nxt], sems[nxt]
            )
        pltpu.sync_copy(bufs[cur], out_hbm.at[pl.ds(base + c * CHUNK, CHUNK)])


def kernel(position_ids, pe):
    idx = position_ids.reshape(NW, NCHUNK, CHUNK).astype(jnp.int32)
    table = pe.reshape(MAX_LEN, D_MODEL)
    out = _gather_rows(idx, table)
    return out.reshape(position_ids.shape + (1, D_MODEL))


# speedup vs baseline: 3.2639x; 1.6002x over previous
import functools

import jax
import jax.numpy as jnp
from jax import lax
from jax.experimental import pallas as pl
from jax.experimental.pallas import tpu as pltpu
from jax.experimental.pallas import tpu_sc as plsc

MAX_LEN = 8192
D_MODEL = 768
BATCH = 4
SEQ = 8192
B_TOT = BATCH * SEQ
NW = 32
B_PER_W = B_TOT // NW
CHUNK = 64
NCHUNK = B_PER_W // CHUNK

_mesh = plsc.VectorSubcoreMesh(core_axis_name="c", subcore_axis_name="s")


@functools.partial(
    pl.kernel,
    mesh=_mesh,
    out_type=jax.ShapeDtypeStruct((BATCH, SEQ, 1, D_MODEL), jnp.float32),
    scratch_types=[
        pltpu.VMEM((NCHUNK, CHUNK), jnp.int32),
        pltpu.VMEM((CHUNK, D_MODEL), jnp.float32),
        pltpu.VMEM((CHUNK, D_MODEL), jnp.float32),
        pltpu.SemaphoreType.DMA,
        pltpu.SemaphoreType.DMA,
    ],
)
def _gather_rows(idx_hbm, table_hbm, out_hbm, idx_v, buf0, buf1, sem0, sem1):
    wid = lax.axis_index("s") * 2 + lax.axis_index("c")
    batch = wid // (NW // BATCH)
    seq_base = (wid % (NW // BATCH)) * B_PER_W
    pltpu.sync_copy(idx_hbm.at[wid], idx_v)

    bufs = (buf0, buf1)
    sems = (sem0, sem1)
    copies = [None, None]
    copies[0] = pltpu.async_copy(table_hbm.at[idx_v.at[0]], bufs[0], sems[0])
    for c in range(NCHUNK):
        cur = c % 2
        nxt = (c + 1) % 2
        copies[cur].wait()
        if c + 1 < NCHUNK:
            copies[nxt] = pltpu.async_copy(
                table_hbm.at[idx_v.at[c + 1]], bufs[nxt], sems[nxt]
            )
        pltpu.sync_copy(
            bufs[cur],
            out_hbm.at[batch, pl.ds(seq_base + c * CHUNK, CHUNK), 0],
        )


def kernel(position_ids, pe):
    idx = position_ids.reshape(NW, NCHUNK, CHUNK).astype(jnp.int32)
    table = pe.reshape(MAX_LEN, D_MODEL)
    return _gather_rows(idx, table)
